# baseline (speedup 1.0000x reference)
import functools

import jax
import jax.numpy as jnp
import numpy as np
from jax import lax
from jax.experimental import pallas as pl
from jax.experimental.pallas import tpu as pltpu
from jax.experimental.pallas import tpu_sc as plsc

N_LEVELS = 16
LOG2_HASHMAP_SIZE = 19
V = 2 ** LOG2_HASHMAP_SIZE
N_FEATURES = 2
COARSEST = 16
FINEST = 512
N_POINTS = 524288

NC, NS = 2, 16
NW = NC * NS
PPT = N_POINTS // NW
C = 512
NCHUNK = PPT // C
NIDX = 8 * C
SLEN = 512
NSTREAM = NIDX // SLEN

_B = float(np.exp((np.log(float(FINEST)) - np.log(float(COARSEST))) / (N_LEVELS - 1)))
RES = [float(np.floor(COARSEST * _B ** l)) for l in range(N_LEVELS)]
MASK = V - 1
P1 = np.int32(np.uint32(2654435761))
P2 = np.int32(np.uint32(805459861))
HI_MASK = np.int32(np.uint32(0xFFFF0000))

DENSE = [(0, 17, 32), (1, 21, 32), (2, 26, 32)]
NGRID = 26 * 26 * 32 + 2048
NDENSE = len(DENSE)


def _body(
    xt_hbm, tp_hbm, res_hbm, out_hbm,
    xv, idxa, idxb, rowsa, rowsb, outb, resm, grid, spt, sema, semb,
):
    wid = lax.axis_index("s") * NC + lax.axis_index("c")
    sid = lax.axis_index("s")
    base = wid * PPT

    pltpu.sync_copy(xt_hbm.at[:, pl.ds(base, PPT)], xv)
    pltpu.sync_copy(res_hbm, resm)

    iota = lax.iota(jnp.int32, 16)

    def stage(l):
        pltpu.sync_copy(
            tp_hbm.at[pl.ds(l * V + sid * (V // NS), V // NS)],
            spt.at[pl.ds(sid * (V // NS), V // NS)],
        )
        plsc.subcore_barrier()

    def frac_coords(cbase, g, res):
        x0 = xv[0, pl.ds(cbase + g * 16, 16)]
        x1 = xv[1, pl.ds(cbase + g * 16, 16)]
        x2 = xv[2, pl.ds(cbase + g * 16, 16)]
        s0 = x0 * res
        s1 = x1 * res
        s2 = x2 * res
        v0 = s0.astype(jnp.int32)
        v1 = s1.astype(jnp.int32)
        v2 = s2.astype(jnp.int32)
        fx = s0 - v0.astype(jnp.float32)
        fy = s1 - v1.astype(jnp.float32)
        fz = s2 - v2.astype(jnp.float32)
        return v0, v1, v2, fx, fy, fz

    def lerp_out(vals_pair, fx, fy, fz, g):
        va = []
        vb = []
        for pair in vals_pair:
            va.append(plsc.bitcast(pair << 16, jnp.float32))
            vb.append(plsc.bitcast(pair & HI_MASK, jnp.float32))
        for f, vals in enumerate((va, vb)):
            c00 = vals[0] + fx * (vals[4] - vals[0])
            c01 = vals[1] + fx * (vals[5] - vals[1])
            c10 = vals[2] + fx * (vals[6] - vals[2])
            c11 = vals[3] + fx * (vals[7] - vals[3])
            cc0 = c00 + fy * (c10 - c00)
            cc1 = c01 + fy * (c11 - c01)
            outb[f, pl.ds(g * 16, 16)] = cc0 + fz * (cc1 - cc0)

    def out_dma(l, cbase):
        pltpu.sync_copy(outb.at[0], out_hbm.at[l, 0, pl.ds(base + cbase, C)])
        pltpu.sync_copy(outb.at[1], out_hbm.at[l, 1, pl.ds(base + cbase, C)])

    for l, r1, r1p in DENSE:
        res = RES[l]
        nplane = r1 * r1p
        nst = (nplane + SLEN - 1) // SLEN

        stage(l)

        def plane(v0, _, r1=r1, r1p=r1p, nplane=nplane, nst=nst):
            def prow(v1, _):
                hxy = v0 ^ (v1 * P1)

                def pz(gz, _):
                    v2 = gz * 16 + iota
                    h = (hxy ^ (v2 * P2)) & MASK
                    pos = v1 * r1p + gz * 16
                    idxa[pos >> 9, pl.ds(pos & 511, 16)] = h
                    return 0

                lax.fori_loop(0, r1p // 16, pz, 0)
                return 0

            lax.fori_loop(0, nst * SLEN // r1p, prow, 0)

            for j in range(nst):
                pltpu.make_async_copy(
                    spt.at[idxa.at[j]],
                    grid.at[pl.ds(v0 * nplane + j * SLEN, SLEN)],
                    sema,
                ).start()
            for j in range(nst):
                pltpu.make_async_copy(
                    spt.at[idxa.at[j]],
                    grid.at[pl.ds(v0 * nplane + j * SLEN, SLEN)],
                    sema,
                ).wait()
            return 0

        lax.fori_loop(0, r1, plane, 0)

        def dchunk(ci, _, l=l, r1=r1, r1p=r1p, res=res):
            cbase = ci * C

            def dp(g, _):
                v0, v1, v2, fx, fy, fz = frac_coords(cbase, g, res)
                bidx = (v0 * r1 + v1) * r1p + v2
                vals = []
                for i in (0, 1):
                    for j in (0, 1):
                        for k in (0, 1):
                            off = i * r1 * r1p + j * r1p + k
                            vals.append(plsc.load_gather(grid, [bidx + off]))
                lerp_out(vals, fx, fy, fz, g)
                return 0

            lax.fori_loop(0, C // 16, dp, 0)
            out_dma(l, cbase)
            return 0

        lax.fori_loop(0, NCHUNK, dchunk, 0)
        plsc.subcore_barrier()

    def p1_fire(ci, idxr, rowsr, sem, res):
        cbase = ci * C

        def p1(g, _):
            v0, v1, v2, _fx, _fy, _fz = frac_coords(cbase, g, res)
            a0 = v0
            a1 = v0 + 1
            b0 = v1 * P1
            b1 = b0 + P1
            c0 = v2 * P2
            c1 = c0 + P2
            t00 = a0 ^ b0
            t01 = a0 ^ b1
            t10 = a1 ^ b0
            t11 = a1 ^ b1
            row = (g >> 5)
            col = (g & 31) * 16
            hs = (
                t00 ^ c0, t00 ^ c1, t01 ^ c0, t01 ^ c1,
                t10 ^ c0, t10 ^ c1, t11 ^ c0, t11 ^ c1,
            )
            for cidx, h in enumerate(hs):
                idxr[cidx * (C // SLEN) + row, pl.ds(col, 16)] = h & MASK
            return 0

        lax.fori_loop(0, C // 16, p1, 0)

        def fire(j, _):
            pltpu.make_async_copy(
                spt.at[idxr.at[j]], rowsr.at[pl.ds(j * SLEN, SLEN)], sem
            ).start()
            return 0

        lax.fori_loop(0, NSTREAM, fire, 0)

    def drain_p2_out(ci, idxr, rowsr, sem, l, res):
        cbase = ci * C

        def drain(j, _):
            pltpu.make_async_copy(
                spt.at[idxr.at[j]], rowsr.at[pl.ds(j * SLEN, SLEN)], sem
            ).wait()
            return 0

        lax.fori_loop(0, NSTREAM, drain, 0)

        def p2(g, _):
            _v0, _v1, _v2, fx, fy, fz = frac_coords(cbase, g, res)
            vals = [rowsr[pl.ds(cidx * C + g * 16, 16)] for cidx in range(8)]
            lerp_out(vals, fx, fy, fz, g)
            return 0

        lax.fori_loop(0, C // 16, p2, 0)
        out_dma(l, cbase)

    def level(l, _):
        res = plsc.load_gather(resm, [jnp.full((16,), l, jnp.int32)])
        stage(l)

        p1_fire(0, idxa, rowsa, sema, res)

        def pair_body(k, _):
            p1_fire(2 * k + 1, idxb, rowsb, semb, res)
            drain_p2_out(2 * k, idxa, rowsa, sema, l, res)

            @pl.when(2 * k + 2 < NCHUNK)
            def _nexta():
                p1_fire(2 * k + 2, idxa, rowsa, sema, res)

            drain_p2_out(2 * k + 1, idxb, rowsb, semb, l, res)
            return 0

        lax.fori_loop(0, NCHUNK // 2, pair_body, 0)
        plsc.subcore_barrier()
        return 0

    lax.fori_loop(NDENSE, N_LEVELS, level, 0)


@jax.jit
def kernel(x, tables):
    xt = x.T
    tb = tables.astype(jnp.bfloat16)
    tu = jax.lax.bitcast_convert_type(tb, jnp.uint16).astype(jnp.uint32)
    tp = jax.lax.bitcast_convert_type((tu[..., 1] << 16) | tu[..., 0], jnp.int32)
    tp = tp.reshape(N_LEVELS * V)
    resarr = jnp.array(RES, dtype=jnp.float32)
    mesh = plsc.VectorSubcoreMesh(core_axis_name="c", subcore_axis_name="s")
    run = functools.partial(
        pl.kernel,
        mesh=mesh,
        compiler_params=pltpu.CompilerParams(
            needs_layout_passes=False, use_tc_tiling_on_sc=False
        ),
        out_type=jax.ShapeDtypeStruct((N_LEVELS, N_FEATURES, N_POINTS), jnp.float32),
        scratch_types=[
            pltpu.VMEM((3, PPT), jnp.float32),
            pltpu.VMEM((NSTREAM, SLEN), jnp.int32),
            pltpu.VMEM((NSTREAM, SLEN), jnp.int32),
            pltpu.VMEM((NIDX,), jnp.int32),
            pltpu.VMEM((NIDX,), jnp.int32),
            pltpu.VMEM((N_FEATURES, C), jnp.float32),
            pltpu.VMEM((N_LEVELS,), jnp.float32),
            pltpu.VMEM((NGRID,), jnp.int32),
            pltpu.VMEM_SHARED((V,), jnp.int32),
            pltpu.SemaphoreType.DMA,
            pltpu.SemaphoreType.DMA,
        ],
    )(_body)
    out = run(xt, tp, resarr)
    return out.transpose(2, 0, 1).reshape(N_POINTS, N_LEVELS * N_FEATURES)


# speedup vs baseline: 1.0169x; 1.0169x over previous
import functools

import jax
import jax.numpy as jnp
import numpy as np
from jax import lax
from jax.experimental import pallas as pl
from jax.experimental.pallas import tpu as pltpu
from jax.experimental.pallas import tpu_sc as plsc

N_LEVELS = 16
LOG2_HASHMAP_SIZE = 19
V = 2 ** LOG2_HASHMAP_SIZE
N_FEATURES = 2
COARSEST = 16
FINEST = 512
N_POINTS = 524288

NC, NS = 2, 16
NW = NC * NS
PPT = N_POINTS // NW
C = 1024
NCHUNK = PPT // C
NIDX = 8 * C
SLEN = 512
NSTREAM = NIDX // SLEN

_B = float(np.exp((np.log(float(FINEST)) - np.log(float(COARSEST))) / (N_LEVELS - 1)))
RES = [float(np.floor(COARSEST * _B ** l)) for l in range(N_LEVELS)]
MASK = V - 1
P1 = np.int32(np.uint32(2654435761))
P2 = np.int32(np.uint32(805459861))
HI_MASK = np.int32(np.uint32(0xFFFF0000))

DENSE = [(0, 17, 32)]
NGRID = 16 * 17 * 32 + 2 * 512
NDENSE = len(DENSE)


def _body(
    xt_hbm, tp_hbm, res_hbm, out_hbm,
    xv, idxa, idxb, rowsa, rowsb, outb, resm, grid, spt, sema, semb,
):
    wid = lax.axis_index("s") * NC + lax.axis_index("c")
    sid = lax.axis_index("s")
    base = wid * PPT

    pltpu.sync_copy(xt_hbm.at[:, pl.ds(base, PPT)], xv)
    pltpu.sync_copy(res_hbm, resm)

    iota = lax.iota(jnp.int32, 16)

    def stage(l):
        pltpu.sync_copy(
            tp_hbm.at[pl.ds(l * V + sid * (V // NS), V // NS)],
            spt.at[pl.ds(sid * (V // NS), V // NS)],
        )
        plsc.subcore_barrier()

    def frac_coords(cbase, g, res):
        x0 = xv[0, pl.ds(cbase + g * 16, 16)]
        x1 = xv[1, pl.ds(cbase + g * 16, 16)]
        x2 = xv[2, pl.ds(cbase + g * 16, 16)]
        s0 = x0 * res
        s1 = x1 * res
        s2 = x2 * res
        v0 = s0.astype(jnp.int32)
        v1 = s1.astype(jnp.int32)
        v2 = s2.astype(jnp.int32)
        fx = s0 - v0.astype(jnp.float32)
        fy = s1 - v1.astype(jnp.float32)
        fz = s2 - v2.astype(jnp.float32)
        return v0, v1, v2, fx, fy, fz

    def lerp_out(vals_pair, fx, fy, fz, g):
        va = []
        vb = []
        for pair in vals_pair:
            va.append(plsc.bitcast(pair << 16, jnp.float32))
            vb.append(plsc.bitcast(pair & HI_MASK, jnp.float32))
        for f, vals in enumerate((va, vb)):
            c00 = vals[0] + fx * (vals[4] - vals[0])
            c01 = vals[1] + fx * (vals[5] - vals[1])
            c10 = vals[2] + fx * (vals[6] - vals[2])
            c11 = vals[3] + fx * (vals[7] - vals[3])
            cc0 = c00 + fy * (c10 - c00)
            cc1 = c01 + fy * (c11 - c01)
            outb[f, pl.ds(g * 16, 16)] = cc0 + fz * (cc1 - cc0)

    def out_dma(l, cbase):
        pltpu.sync_copy(outb.at[0], out_hbm.at[l, 0, pl.ds(base + cbase, C)])
        pltpu.sync_copy(outb.at[1], out_hbm.at[l, 1, pl.ds(base + cbase, C)])

    for l, r1, r1p in DENSE:
        res = RES[l]
        nplane = r1 * r1p
        nst = (nplane + SLEN - 1) // SLEN

        stage(l)

        def plane(v0, _, r1=r1, r1p=r1p, nplane=nplane, nst=nst):
            def prow(v1, _):
                hxy = v0 ^ (v1 * P1)

                def pz(gz, _):
                    v2 = gz * 16 + iota
                    h = (hxy ^ (v2 * P2)) & MASK
                    pos = v1 * r1p + gz * 16
                    idxa[pos >> 9, pl.ds(pos & 511, 16)] = h
                    return 0

                lax.fori_loop(0, r1p // 16, pz, 0)
                return 0

            lax.fori_loop(0, nst * SLEN // r1p, prow, 0)

            for j in range(nst):
                pltpu.make_async_copy(
                    spt.at[idxa.at[j]],
                    grid.at[pl.ds(v0 * nplane + j * SLEN, SLEN)],
                    sema,
                ).start()
            for j in range(nst):
                pltpu.make_async_copy(
                    spt.at[idxa.at[j]],
                    grid.at[pl.ds(v0 * nplane + j * SLEN, SLEN)],
                    sema,
                ).wait()
            return 0

        lax.fori_loop(0, r1, plane, 0)

        def dchunk(ci, _, l=l, r1=r1, r1p=r1p, res=res):
            cbase = ci * C

            def dp(g, _):
                v0, v1, v2, fx, fy, fz = frac_coords(cbase, g, res)
                bidx = (v0 * r1 + v1) * r1p + v2
                vals = []
                for i in (0, 1):
                    for j in (0, 1):
                        for k in (0, 1):
                            off = i * r1 * r1p + j * r1p + k
                            vals.append(plsc.load_gather(grid, [bidx + off]))
                lerp_out(vals, fx, fy, fz, g)
                return 0

            lax.fori_loop(0, C // 16, dp, 0)
            out_dma(l, cbase)
            return 0

        lax.fori_loop(0, NCHUNK, dchunk, 0)
        plsc.subcore_barrier()

    def p1_fire(ci, idxr, rowsr, sem, res):
        cbase = ci * C

        def p1(g, _):
            v0, v1, v2, _fx, _fy, _fz = frac_coords(cbase, g, res)
            a0 = v0
            a1 = v0 + 1
            b0 = v1 * P1
            b1 = b0 + P1
            c0 = v2 * P2
            c1 = c0 + P2
            t00 = a0 ^ b0
            t01 = a0 ^ b1
            t10 = a1 ^ b0
            t11 = a1 ^ b1
            row = (g >> 5)
            col = (g & 31) * 16
            hs = (
                t00 ^ c0, t00 ^ c1, t01 ^ c0, t01 ^ c1,
                t10 ^ c0, t10 ^ c1, t11 ^ c0, t11 ^ c1,
            )
            for cidx, h in enumerate(hs):
                idxr[cidx * (C // SLEN) + row, pl.ds(col, 16)] = h & MASK
            return 0

        lax.fori_loop(0, C // 16, p1, 0)

        def fire(j, _):
            pltpu.make_async_copy(
                spt.at[idxr.at[j]], rowsr.at[pl.ds(j * SLEN, SLEN)], sem
            ).start()
            return 0

        lax.fori_loop(0, NSTREAM, fire, 0)

    def drain_p2_out(ci, idxr, rowsr, sem, l, res):
        cbase = ci * C

        def drain(j, _):
            pltpu.make_async_copy(
                spt.at[idxr.at[j]], rowsr.at[pl.ds(j * SLEN, SLEN)], sem
            ).wait()
            return 0

        lax.fori_loop(0, NSTREAM, drain, 0)

        def p2(g, _):
            _v0, _v1, _v2, fx, fy, fz = frac_coords(cbase, g, res)
            vals = [rowsr[pl.ds(cidx * C + g * 16, 16)] for cidx in range(8)]
            lerp_out(vals, fx, fy, fz, g)
            return 0

        lax.fori_loop(0, C // 16, p2, 0)
        out_dma(l, cbase)

    def level(l, _):
        res = plsc.load_gather(resm, [jnp.full((16,), l, jnp.int32)])
        stage(l)

        p1_fire(0, idxa, rowsa, sema, res)

        def pair_body(k, _):
            p1_fire(2 * k + 1, idxb, rowsb, semb, res)
            drain_p2_out(2 * k, idxa, rowsa, sema, l, res)

            @pl.when(2 * k + 2 < NCHUNK)
            def _nexta():
                p1_fire(2 * k + 2, idxa, rowsa, sema, res)

            drain_p2_out(2 * k + 1, idxb, rowsb, semb, l, res)
            return 0

        lax.fori_loop(0, NCHUNK // 2, pair_body, 0)
        plsc.subcore_barrier()
        return 0

    lax.fori_loop(NDENSE, N_LEVELS, level, 0)


@jax.jit
def kernel(x, tables):
    xt = x.T
    tb = tables.astype(jnp.bfloat16)
    tu = jax.lax.bitcast_convert_type(tb, jnp.uint16).astype(jnp.uint32)
    tp = jax.lax.bitcast_convert_type((tu[..., 1] << 16) | tu[..., 0], jnp.int32)
    tp = tp.reshape(N_LEVELS * V)
    resarr = jnp.array(RES, dtype=jnp.float32)
    mesh = plsc.VectorSubcoreMesh(core_axis_name="c", subcore_axis_name="s")
    run = functools.partial(
        pl.kernel,
        mesh=mesh,
        compiler_params=pltpu.CompilerParams(
            needs_layout_passes=False, use_tc_tiling_on_sc=False
        ),
        out_type=jax.ShapeDtypeStruct((N_LEVELS, N_FEATURES, N_POINTS), jnp.float32),
        scratch_types=[
            pltpu.VMEM((3, PPT), jnp.float32),
            pltpu.VMEM((NSTREAM, SLEN), jnp.int32),
            pltpu.VMEM((NSTREAM, SLEN), jnp.int32),
            pltpu.VMEM((NIDX,), jnp.int32),
            pltpu.VMEM((NIDX,), jnp.int32),
            pltpu.VMEM((N_FEATURES, C), jnp.float32),
            pltpu.VMEM((N_LEVELS,), jnp.float32),
            pltpu.VMEM((NGRID,), jnp.int32),
            pltpu.VMEM_SHARED((V,), jnp.int32),
            pltpu.SemaphoreType.DMA,
            pltpu.SemaphoreType.DMA,
        ],
    )(_body)
    out = run(xt, tp, resarr)
    return out.transpose(2, 0, 1).reshape(N_POINTS, N_LEVELS * N_FEATURES)
